# baseline (speedup 1.0000x reference)
import jax
import jax.numpy as jnp
from jax import lax
from jax.experimental import pallas as pl
from jax.experimental.pallas import tpu as pltpu

_ROT_A = (13, 15, 26, 6)
_ROT_B = (17, 29, 16, 24)


def _rotl(x, d):
    return (x << jnp.uint32(d)) | (x >> jnp.uint32(32 - d))


def _four_rounds(x0, x1, rots):
    for r in rots:
        x0 = x0 + x1
        x1 = _rotl(x1, r)
        x1 = x0 ^ x1
    return x0, x1


def _threefry_bits(lo, ks1):
    ks2 = ks1 ^ jnp.uint32(0x1BD11BDA)
    x1 = lo + ks1
    x0 = x1
    x1 = _rotl(x1, _ROT_A[0]) ^ x0
    for r in _ROT_A[1:]:
        x0 = x0 + x1
        x1 = _rotl(x1, r)
        x1 = x0 ^ x1
    x0 = x0 + ks1
    x1 = x1 + ks2 + jnp.uint32(1)
    x0, x1 = _four_rounds(x0, x1, _ROT_B)
    x0 = x0 + ks2
    x1 = x1 + jnp.uint32(2)
    x0, x1 = _four_rounds(x0, x1, _ROT_A)
    x1 = x1 + ks1 + jnp.uint32(3)
    x0, x1 = _four_rounds(x0, x1, _ROT_B)
    x0 = x0 + ks1
    x1 = x1 + ks2 + jnp.uint32(4)
    x0, x1 = _four_rounds(x0, x1, _ROT_A)
    x0 = x0 + ks2
    x1 = x1 + jnp.uint32(5)
    return x0 ^ x1


def _neg_gumbel_from_bits(bits):
    tiny = jnp.float32(jnp.finfo(jnp.float32).tiny)
    fb = (bits >> jnp.uint32(9)) | jnp.uint32(0x3F800000)
    f = lax.bitcast_convert_type(fb, jnp.float32) - jnp.float32(1.0)
    return jnp.log(-jnp.log(f + tiny))


def _body(prow_ref, pcol_ref, dist_ref, out_ref):
    br, b = dist_ref.shape
    i = pl.program_id(0)
    d = dist_ref[...]
    mask = prow_ref[...] == pcol_ref[...]
    col = lax.broadcasted_iota(jnp.int32, (br, b), 1)
    row = lax.broadcasted_iota(jnp.int32, (br, b), 0) + i * br
    lo = (row * b + col).astype(jnp.uint32)
    ks1 = jnp.where(mask, jnp.uint32(123), jnp.uint32(456))
    nl = _neg_gumbel_from_bits(_threefry_bits(lo, ks1))
    ninf = jnp.float32(-jnp.inf)
    pinf = jnp.float32(jnp.inf)
    sp = jnp.where(mask, d - nl, ninf)
    sn = jnp.where(mask, pinf, d + nl)
    mp = jnp.max(sp, axis=1, keepdims=True)
    mn = jnp.min(sn, axis=1, keepdims=True)
    pos = jnp.max(jnp.where(sp == mp, d, ninf), axis=1, keepdims=True)
    neg = jnp.max(jnp.where(sn == mn, d, ninf), axis=1, keepdims=True)
    neg = jnp.where(mn == pinf, d[:, 0:1], neg)
    x = pos - neg
    out_ref[...] = jnp.logaddexp(x, jnp.float32(0.0))


def kernel(dist, pids):
    b = dist.shape[0]
    br = 64
    grid = (b // br,)
    prow = pids.reshape(b, 1)
    pcol = pids.reshape(1, b)
    out = pl.pallas_call(
        _body,
        grid=grid,
        in_specs=[
            pl.BlockSpec((br, 1), lambda i: (i, 0)),
            pl.BlockSpec((1, b), lambda i: (0, 0)),
            pl.BlockSpec((br, b), lambda i: (i, 0)),
        ],
        out_specs=pl.BlockSpec((br, 1), lambda i: (i, 0)),
        out_shape=jax.ShapeDtypeStruct((b, 1), jnp.float32),
        compiler_params=pltpu.CompilerParams(
            dimension_semantics=("parallel",)),
    )(prow, pcol, dist)
    return out.reshape(b)


# speedup vs baseline: 1.0191x; 1.0191x over previous
import jax
import jax.numpy as jnp
from jax import lax
from jax.experimental import pallas as pl
from jax.experimental.pallas import tpu as pltpu

_ROT_A = (13, 15, 26, 6)
_ROT_B = (17, 29, 16, 24)


def _rotl(x, d):
    return (x << jnp.uint32(d)) | (x >> jnp.uint32(32 - d))


def _four_rounds(x0, x1, rots):
    for r in rots:
        x0 = x0 + x1
        x1 = _rotl(x1, r)
        x1 = x0 ^ x1
    return x0, x1


def _threefry_bits(lo, ks1):
    ks2 = ks1 ^ jnp.uint32(0x1BD11BDA)
    x1 = lo + ks1
    x0 = x1
    x1 = _rotl(x1, _ROT_A[0]) ^ x0
    for r in _ROT_A[1:]:
        x0 = x0 + x1
        x1 = _rotl(x1, r)
        x1 = x0 ^ x1
    x0 = x0 + ks1
    x1 = x1 + ks2 + jnp.uint32(1)
    x0, x1 = _four_rounds(x0, x1, _ROT_B)
    x0 = x0 + ks2
    x1 = x1 + jnp.uint32(2)
    x0, x1 = _four_rounds(x0, x1, _ROT_A)
    x1 = x1 + ks1 + jnp.uint32(3)
    x0, x1 = _four_rounds(x0, x1, _ROT_B)
    x0 = x0 + ks1
    x1 = x1 + ks2 + jnp.uint32(4)
    x0, x1 = _four_rounds(x0, x1, _ROT_A)
    x0 = x0 + ks2
    x1 = x1 + jnp.uint32(5)
    return x0 ^ x1


def _neg_gumbel_from_bits(bits):
    tiny = jnp.float32(jnp.finfo(jnp.float32).tiny)
    fb = (bits >> jnp.uint32(9)) | jnp.uint32(0x3F800000)
    f = lax.bitcast_convert_type(fb, jnp.float32) - jnp.float32(1.0)
    return jnp.log(-jnp.log(f + tiny))


def _body(prow_ref, pcol_ref, dist_ref, out_ref):
    br, b = dist_ref.shape
    i = pl.program_id(0)
    d = dist_ref[...]
    mask = prow_ref[...] == pcol_ref[...]
    col = lax.broadcasted_iota(jnp.int32, (br, b), 1)
    row = lax.broadcasted_iota(jnp.int32, (br, b), 0) + i * br
    lo = (row * b + col).astype(jnp.uint32)
    ks1 = jnp.where(mask, jnp.uint32(123), jnp.uint32(456))
    nl = _neg_gumbel_from_bits(_threefry_bits(lo, ks1))
    ninf = jnp.float32(-jnp.inf)
    pinf = jnp.float32(jnp.inf)
    sp = jnp.where(mask, d - nl, ninf)
    sn = jnp.where(mask, pinf, d + nl)
    mp = jnp.max(sp, axis=1, keepdims=True)
    mn = jnp.min(sn, axis=1, keepdims=True)
    pos = jnp.max(jnp.where(sp == mp, d, ninf), axis=1, keepdims=True)
    neg = jnp.max(jnp.where(sn == mn, d, ninf), axis=1, keepdims=True)
    neg = jnp.where(mn == pinf, d[:, 0:1], neg)
    x = pos - neg
    out_ref[...] = jnp.logaddexp(x, jnp.float32(0.0))


def kernel(dist, pids):
    b = dist.shape[0]
    br = 256
    grid = (b // br,)
    prow = pids.reshape(b, 1)
    pcol = pids.reshape(1, b)
    out = pl.pallas_call(
        _body,
        grid=grid,
        in_specs=[
            pl.BlockSpec((br, 1), lambda i: (i, 0)),
            pl.BlockSpec((1, b), lambda i: (0, 0)),
            pl.BlockSpec((br, b), lambda i: (i, 0)),
        ],
        out_specs=pl.BlockSpec((br, 1), lambda i: (i, 0)),
        out_shape=jax.ShapeDtypeStruct((b, 1), jnp.float32),
        compiler_params=pltpu.CompilerParams(
            dimension_semantics=("parallel",)),
    )(prow, pcol, dist)
    return out.reshape(b)
